# Initial kernel scaffold; baseline (speedup 1.0000x reference)
import jax
import jax.numpy as jnp
from jax.experimental import pallas as pl


def kernel(features, edge_index, edge_weight, W, bias, prelu_a):
    raise NotImplementedError("write your pallas kernel here")


# speedup vs baseline: 2.8507x; 2.8507x over previous
import functools

import jax
import jax.numpy as jnp
from jax import lax
from jax.experimental import pallas as pl
from jax.experimental.pallas import tpu as pltpu
from jax.experimental.pallas import tpu_sc as plsc

N_NODES = 10000
D_IN = 256
D_OUT = 256
DH = 128

NC = 2
NS = 16
K = 128
CHT = 80
E_PAD = NS * CHT * K

MB = 1000
N_PAD = 10240
RPT = N_PAD // NS
RB = 128


def _mm_body(x_ref, w_ref, h_ref, hs_ref):
    h = lax.dot_general(x_ref[...], w_ref[...],
                        (((1,), (1,)), ((), ())),
                        preferred_element_type=jnp.float32)
    h_ref[...] = h
    hs_ref[0] = h[:, :DH]
    hs_ref[1] = h[:, DH:]


def _matmul(x, w):
    return pl.pallas_call(
        _mm_body,
        grid=(N_NODES // MB,),
        in_specs=[
            pl.BlockSpec((MB, D_IN), lambda i: (i, 0)),
            pl.BlockSpec((D_OUT, D_IN), lambda i: (0, 0)),
        ],
        out_specs=[
            pl.BlockSpec((MB, D_OUT), lambda i: (i, 0)),
            pl.BlockSpec((2, MB, DH), lambda i: (0, i, 0)),
        ],
        out_shape=[
            jax.ShapeDtypeStruct((N_NODES, D_OUT), jnp.float32),
            jax.ShapeDtypeStruct((2, N_NODES, DH), jnp.float32),
        ],
    )(x, w)


def _sc_agg(hs_flat, col_t, row_t, w_t):
    mesh = plsc.VectorSubcoreMesh(core_axis_name="c", subcore_axis_name="s")

    @functools.partial(
        pl.kernel,
        out_type=jax.ShapeDtypeStruct((NC, N_PAD, DH), jnp.float32),
        mesh=mesh,
        scratch_types=[
            pltpu.VMEM((CHT, K), jnp.int32),
            pltpu.VMEM((CHT, K), jnp.int32),
            pltpu.VMEM((CHT, K), jnp.float32),
            pltpu.VMEM((K, DH), jnp.float32),
            pltpu.VMEM_SHARED((N_PAD, DH), jnp.float32),
        ],
    )
    def body(hs_hbm, col_hbm, row_hbm, w_hbm, out_hbm,
             col_v, row_v, w_v, gbuf, acc):
        cid = lax.axis_index("c")
        sid = lax.axis_index("s")

        pltpu.sync_copy(col_hbm.at[sid], col_v)
        pltpu.sync_copy(row_hbm.at[sid], row_v)
        pltpu.sync_copy(w_hbm.at[sid], w_v)

        offv = jnp.full((16,), cid * N_NODES, dtype=jnp.int32)

        @pl.loop(0, CHT)
        def _(ci):
            for j in range(K // 16):
                sl = pl.ds(j * 16, 16)
                col_v[ci, sl] = col_v[ci, sl] + offv

        @pl.loop(0, RB)
        def _(r):
            for j in range(DH // 16):
                gbuf[r, pl.ds(j * 16, 16)] = jnp.zeros((16,), jnp.float32)

        for b in range(RPT // RB):
            pltpu.sync_copy(gbuf, acc.at[pl.ds(sid * RPT + b * RB, RB)])
        plsc.subcore_barrier()

        @pl.loop(0, CHT)
        def _(ci):
            pltpu.sync_copy(hs_hbm.at[col_v.at[ci]], gbuf)

            @pl.loop(0, K // 16)
            def _(g):
                wvec = w_v[ci, pl.ds(g * 16, 16)]
                for e16 in range(16):
                    wb = jnp.full((16,), wvec[e16], dtype=jnp.float32)
                    for j in range(DH // 16):
                        sl = pl.ds(j * 16, 16)
                        e = g * 16 + e16
                        gbuf[e, sl] = gbuf[e, sl] * wb

            pltpu.sync_copy(gbuf, acc.at[row_v.at[ci]], add=True)
        plsc.subcore_barrier()

        for b in range(RPT // RB):
            start = sid * RPT + b * RB
            pltpu.sync_copy(acc.at[pl.ds(start, RB)],
                            out_hbm.at[cid].at[pl.ds(start, RB)])

    return body(hs_flat, col_t, row_t, w_t)


def _ep_body(agg_ref, b_ref, a_ref, o_ref):
    m = jnp.concatenate([agg_ref[0], agg_ref[1]], axis=1) + b_ref[...]
    a = a_ref[0, 0]
    o_ref[...] = jnp.where(m >= 0.0, m, a * m)


def _epilogue(agg, bias, prelu_a):
    return pl.pallas_call(
        _ep_body,
        grid=(N_NODES // MB,),
        in_specs=[
            pl.BlockSpec((2, MB, DH), lambda i: (0, i, 0)),
            pl.BlockSpec((1, D_OUT), lambda i: (0, 0)),
            pl.BlockSpec(memory_space=pltpu.SMEM),
        ],
        out_specs=pl.BlockSpec((MB, D_OUT), lambda i: (i, 0)),
        out_shape=jax.ShapeDtypeStruct((N_NODES, D_OUT), jnp.float32),
    )(agg, bias.reshape(1, D_OUT), prelu_a.reshape(1, 1))


def kernel(features, edge_index, edge_weight, W, bias, prelu_a):
    x = features[0]
    row = edge_index[0].astype(jnp.int32)
    col = edge_index[1].astype(jnp.int32)
    w = edge_weight.astype(jnp.float32)

    n_e = row.shape[0]
    pad = E_PAD - n_e
    row_t = jnp.pad(row, (0, pad)).reshape(NS, CHT, K)
    col_t = jnp.pad(col, (0, pad)).reshape(NS, CHT, K)
    w_t = jnp.pad(w, (0, pad)).reshape(NS, CHT, K)

    hidden, hs = _matmul(x, W)
    agg = _sc_agg(hs.reshape(2 * N_NODES, DH), col_t, row_t, w_t)
    act = _epilogue(agg, bias, prelu_a)
    return (act[None], hidden[None])
